# baseline (speedup 1.0000x reference)
import functools

import jax
import jax.numpy as jnp
from jax import lax
from jax.experimental import pallas as pl
from jax.experimental.pallas import tpu as pltpu
from jax.experimental.pallas import tpu_sc as plsc

NC = 2
NS = 16
NW = NC * NS

BB = 4096 // 128
TT = 200

NROW = 1000000
LASTW = NROW // 128


def _transpose_table(tableT):
    mesh = plsc.VectorSubcoreMesh(core_axis_name="c", subcore_axis_name="s")

    @functools.partial(
        pl.kernel,
        out_type=jax.ShapeDtypeStruct((500000, 128), jnp.float32),
        mesh=mesh,
        scratch_types=[
            pltpu.VMEM((2, 64, 128), jnp.float32),
            pltpu.VMEM((2, 64, 130), jnp.float32),
            pltpu.SemaphoreType.DMA((2,)),
            pltpu.SemaphoreType.DMA((2,)),
        ],
        compiler_params=pltpu.CompilerParams(
            use_tc_tiling_on_sc=True,
            needs_layout_passes=False,
            disable_bounds_checks=True,
        ),
    )
    def ka(tt_hbm, out_hbm, vbuf, wbuf, gsem, wsem):
        wid = lax.axis_index("s") * NC + lax.axis_index("c")
        lane = lax.iota(jnp.int32, 16)
        wv = [lax.shift_right_logical(lane + kk * 16, 1) for kk in range(8)]
        hbase = (lane & 1) * 64
        nwin = LASTW // NW + 1

        def rdwin(g, s):
            return pltpu.make_async_copy(
                tt_hbm.at[:, pl.ds((g * NW + wid) * 128, 128)],
                vbuf.at[s],
                gsem.at[s],
            )

        def wr_full(g, s):
            return pltpu.make_async_copy(
                wbuf.at[s, :, pl.ds(0, 128)],
                out_hbm.at[pl.ds((g * NW + wid) * 64, 64)],
                wsem.at[s],
            )

        def wr_tail(s):
            return pltpu.make_async_copy(
                wbuf.at[s, pl.ds(0, 32), pl.ds(0, 128)],
                out_hbm.at[pl.ds(LASTW * 64, 32)],
                wsem.at[s],
            )

        def wr_start(g, s):
            win = g * NW + wid

            @pl.when(win < LASTW)
            def _():
                wr_full(g, s).start()

            @pl.when(win == LASTW)
            def _():
                wr_tail(s).start()

        def wr_wait(g, s):
            win = g * NW + wid

            @pl.when(win < LASTW)
            def _():
                wr_full(g, s).wait()

            @pl.when(win == LASTW)
            def _():
                wr_tail(s).wait()

        def transpose(s):
            for kk in range(8):
                for c in range(64):
                    v = vbuf[s, c, pl.ds(kk * 16, 16)]
                    plsc.store_scatter(
                        wbuf.at[s], [wv[kk], hbase + c], v
                    )

        @pl.when(wid <= LASTW)
        def _():
            rdwin(0, 0).start()

        def body(g, carry):
            s = lax.rem(g, 2)
            sn = 1 - s
            inb = (g * NW + wid) <= LASTW

            @pl.when(((g + 1) * NW + wid) <= LASTW)
            def _():
                rdwin(g + 1, sn).start()

            @pl.when(inb)
            def _():
                rdwin(g, s).wait()

                @pl.when(g >= 2)
                def _():
                    wr_wait(g - 2, s)

                transpose(s)
                wr_start(g, s)

            return carry

        lax.fori_loop(0, nwin, body, 0, unroll=False)

        @pl.when(((nwin - 2) * NW + wid) <= LASTW)
        def _():
            wr_wait(nwin - 2, lax.rem(nwin - 2, 2))

        @pl.when(((nwin - 1) * NW + wid) <= LASTW)
        def _():
            wr_wait(nwin - 1, lax.rem(nwin - 1, 2))

    return ka(tableT)


def _fused_gather(xr, tablew):
    mesh = plsc.VectorSubcoreMesh(core_axis_name="c", subcore_axis_name="s")

    @functools.partial(
        pl.kernel,
        out_type=jax.ShapeDtypeStruct((TT, 8, BB, 8, 128), jnp.float32),
        mesh=mesh,
        scratch_types=[
            pltpu.VMEM((128 * TT,), jnp.int32),
            pltpu.VMEM((2, 128), jnp.int32),
            pltpu.VMEM((2, 128), jnp.int32),
            pltpu.VMEM((2, 128, 129), jnp.float32),
            pltpu.VMEM((2, 8, 8, 128), jnp.float32),
            pltpu.SemaphoreType.DMA((2,)),
            pltpu.SemaphoreType.DMA((2,)),
        ],
        compiler_params=pltpu.CompilerParams(
            use_tc_tiling_on_sc=True, needs_layout_passes=False
        ),
    )
    def kb(x_hbm, tw_hbm, out_hbm, xv, widx, hoff, rows, obuf, gsem, wsem):
        wid = lax.axis_index("s") * NC + lax.axis_index("c")
        pltpu.sync_copy(x_hbm.at[pl.ds(wid * (128 * TT), 128 * TT)], xv)

        lane = lax.iota(jnp.int32, 16)
        lane200 = lane * TT
        rowv = [lane + kk * 16 for kk in range(8)]

        def build_idx(t, s):
            for kk in range(8):
                v = plsc.load_gather(xv, [lane200 + (kk * 16 * TT + t)])
                widx[s, pl.ds(kk * 16, 16)] = lax.shift_right_logical(v, 1)
                hoff[s, pl.ds(kk * 16, 16)] = (v & 1) * 64

        def gather(s):
            return pltpu.make_async_copy(
                tw_hbm.at[widx.at[s]],
                rows.at[s, :, pl.ds(0, 128)],
                gsem.at[s],
            )

        def transpose(s):
            for kk in range(8):
                hv = hoff[s, pl.ds(kk * 16, 16)]
                for tc in range(8):
                    for ci in range(8):
                        v = plsc.load_gather(
                            rows.at[s], [rowv[kk], hv + (tc * 8 + ci)]
                        )
                        obuf[s, tc, ci, pl.ds(kk * 16, 16)] = v

        def writeback(t, s):
            return pltpu.make_async_copy(
                obuf.at[s], out_hbm.at[t, :, wid], wsem.at[s]
            )

        build_idx(0, 0)
        gather(0).start()

        def body(t, carry):
            s = lax.rem(t, 2)
            sn = 1 - s

            @pl.when(t < TT - 1)
            def _():
                build_idx(t + 1, sn)

            gather(s).wait()

            @pl.when(t < TT - 1)
            def _():
                gather(sn).start()

            @pl.when(t >= 2)
            def _():
                writeback(t - 2, s).wait()

            transpose(s)
            writeback(t, s).start()
            return carry

        lax.fori_loop(0, TT, body, 0, unroll=False)
        writeback(TT - 2, 0).wait()
        writeback(TT - 1, 1).wait()

    return kb(xr, tablew)


def kernel(x, table):
    xr = x.reshape(x.size).astype(jnp.int32)
    tablew = _transpose_table(table.T)
    out5 = _fused_gather(xr, tablew)
    out = out5.transpose(2, 4, 0, 1, 3).reshape(4096, 200, 64)
    return out


# speedup vs baseline: 4.8715x; 4.8715x over previous
import functools

import jax
import jax.numpy as jnp
from jax import lax
from jax.experimental import pallas as pl
from jax.experimental.pallas import tpu as pltpu
from jax.experimental.pallas import tpu_sc as plsc

NC = 2
NS = 16
NW = NC * NS

BB = 4096 // 128
TT = 200

NROW = 1000000
LASTW = NROW // 128


def _transpose_table(tableT):
    mesh = plsc.VectorSubcoreMesh(core_axis_name="c", subcore_axis_name="s")

    @functools.partial(
        pl.kernel,
        out_type=jax.ShapeDtypeStruct((500000, 128), jnp.float32),
        mesh=mesh,
        scratch_types=[
            pltpu.VMEM((2, 64, 128), jnp.float32),
            pltpu.VMEM((2, 64, 130), jnp.float32),
            pltpu.SemaphoreType.DMA((2,)),
            pltpu.SemaphoreType.DMA((2,)),
        ],
        compiler_params=pltpu.CompilerParams(
            use_tc_tiling_on_sc=True,
            needs_layout_passes=False,
            disable_bounds_checks=True,
        ),
    )
    def ka(tt_hbm, out_hbm, vbuf, wbuf, gsem, wsem):
        wid = lax.axis_index("s") * NC + lax.axis_index("c")
        lane = lax.iota(jnp.int32, 16)
        wv = [lax.shift_right_logical(lane + kk * 16, 1) for kk in range(8)]
        hbase = (lane & 1) * 64
        nwin = LASTW // NW + 1

        def rdwin(g, s):
            return pltpu.make_async_copy(
                tt_hbm.at[:, pl.ds((g * NW + wid) * 128, 128)],
                vbuf.at[s],
                gsem.at[s],
            )

        def wr_full(g, s):
            return pltpu.make_async_copy(
                wbuf.at[s, :, pl.ds(0, 128)],
                out_hbm.at[pl.ds((g * NW + wid) * 64, 64)],
                wsem.at[s],
            )

        def wr_tail(s):
            return pltpu.make_async_copy(
                wbuf.at[s, pl.ds(0, 32), pl.ds(0, 128)],
                out_hbm.at[pl.ds(LASTW * 64, 32)],
                wsem.at[s],
            )

        def wr_start(g, s):
            win = g * NW + wid

            @pl.when(win < LASTW)
            def _():
                wr_full(g, s).start()

            @pl.when(win == LASTW)
            def _():
                wr_tail(s).start()

        def wr_wait(g, s):
            win = g * NW + wid

            @pl.when(win < LASTW)
            def _():
                wr_full(g, s).wait()

            @pl.when(win == LASTW)
            def _():
                wr_tail(s).wait()

        def transpose(s):
            for kk in range(8):
                wvk = wv[kk]

                @functools.partial(plsc.parallel_loop, 0, 64, unroll=8)
                def _(c):
                    v = vbuf[s, c, pl.ds(kk * 16, 16)]
                    plsc.store_scatter(wbuf.at[s], [wvk, hbase + c], v)

        @pl.when(wid <= LASTW)
        def _():
            rdwin(0, 0).start()

        def body(g, carry):
            s = lax.rem(g, 2)
            sn = 1 - s
            inb = (g * NW + wid) <= LASTW

            @pl.when(((g + 1) * NW + wid) <= LASTW)
            def _():
                rdwin(g + 1, sn).start()

            @pl.when(inb)
            def _():
                rdwin(g, s).wait()

                @pl.when(g >= 2)
                def _():
                    wr_wait(g - 2, s)

                transpose(s)
                wr_start(g, s)

            return carry

        lax.fori_loop(0, nwin, body, 0, unroll=False)

        @pl.when(((nwin - 2) * NW + wid) <= LASTW)
        def _():
            wr_wait(nwin - 2, lax.rem(nwin - 2, 2))

        @pl.when(((nwin - 1) * NW + wid) <= LASTW)
        def _():
            wr_wait(nwin - 1, lax.rem(nwin - 1, 2))

    return ka(tableT)


def _fused_gather(xr, tablew):
    mesh = plsc.VectorSubcoreMesh(core_axis_name="c", subcore_axis_name="s")

    @functools.partial(
        pl.kernel,
        out_type=jax.ShapeDtypeStruct((TT, 8, BB, 8, 128), jnp.float32),
        mesh=mesh,
        scratch_types=[
            pltpu.VMEM((128 * TT,), jnp.int32),
            pltpu.VMEM((2, 128), jnp.int32),
            pltpu.VMEM((2, 128), jnp.int32),
            pltpu.VMEM((2, 128, 129), jnp.float32),
            pltpu.VMEM((2, 8, 8, 128), jnp.float32),
            pltpu.SemaphoreType.DMA((2,)),
            pltpu.SemaphoreType.DMA((2,)),
        ],
        compiler_params=pltpu.CompilerParams(
            use_tc_tiling_on_sc=True, needs_layout_passes=False
        ),
    )
    def kb(x_hbm, tw_hbm, out_hbm, xv, widx, hoff, rows, obuf, gsem, wsem):
        wid = lax.axis_index("s") * NC + lax.axis_index("c")
        pltpu.sync_copy(x_hbm.at[pl.ds(wid * (128 * TT), 128 * TT)], xv)

        lane = lax.iota(jnp.int32, 16)
        lane200 = lane * TT
        rowv = [lane + kk * 16 for kk in range(8)]

        def build_idx(t, s):
            for kk in range(8):
                v = plsc.load_gather(xv, [lane200 + (kk * 16 * TT + t)])
                widx[s, pl.ds(kk * 16, 16)] = lax.shift_right_logical(v, 1)
                hoff[s, pl.ds(kk * 16, 16)] = (v & 1) * 64

        def gather(s):
            return pltpu.make_async_copy(
                tw_hbm.at[widx.at[s]],
                rows.at[s, :, pl.ds(0, 128)],
                gsem.at[s],
            )

        def transpose(s):
            for kk in range(8):
                hv = hoff[s, pl.ds(kk * 16, 16)]
                rv = rowv[kk]

                @functools.partial(plsc.parallel_loop, 0, 64, unroll=8)
                def _(c):
                    v = plsc.load_gather(rows.at[s], [rv, hv + c])
                    obuf[s, lax.div(c, 8), lax.rem(c, 8),
                         pl.ds(kk * 16, 16)] = v

        def writeback(t, s):
            return pltpu.make_async_copy(
                obuf.at[s], out_hbm.at[t, :, wid], wsem.at[s]
            )

        build_idx(0, 0)
        gather(0).start()

        def body(t, carry):
            s = lax.rem(t, 2)
            sn = 1 - s

            @pl.when(t < TT - 1)
            def _():
                build_idx(t + 1, sn)

            gather(s).wait()

            @pl.when(t < TT - 1)
            def _():
                gather(sn).start()

            @pl.when(t >= 2)
            def _():
                writeback(t - 2, s).wait()

            transpose(s)
            writeback(t, s).start()
            return carry

        lax.fori_loop(0, TT, body, 0, unroll=False)
        writeback(TT - 2, 0).wait()
        writeback(TT - 1, 1).wait()

    return kb(xr, tablew)


def kernel(x, table):
    xr = x.reshape(x.size).astype(jnp.int32)
    tablew = _transpose_table(table.T)
    out5 = _fused_gather(xr, tablew)
    out = out5.transpose(2, 4, 0, 1, 3).reshape(4096, 200, 64)
    return out
